# baseline (speedup 1.0000x reference)
import functools

import jax
import jax.numpy as jnp
from jax import lax
from jax.experimental import pallas as pl
from jax.experimental.pallas import tpu as pltpu
from jax.experimental.pallas import tpu_sc as plsc

BATCH = 4096
FIELDS = 26
PAIRS = FIELDS // 2
DIM = 64
NW = 32
BPW = BATCH // NW
HALF = BATCH // 2
HPW = HALF // NW

_mesh = plsc.VectorSubcoreMesh(core_axis_name="c", subcore_axis_name="s")


def _make_sc(h):
    @functools.partial(
        pl.kernel,
        mesh=_mesh,
        compiler_params=pltpu.CompilerParams(use_tc_tiling_on_sc=False,
                                             needs_layout_passes=False),
        out_type=[
            jax.ShapeDtypeStruct((PAIRS, HALF, 2 * DIM), jnp.float32),
            jax.ShapeDtypeStruct((NW // 2, BPW), jnp.float32),
        ],
        scratch_types=[
            pltpu.VMEM((FIELDS, BPW), jnp.int32),
            pltpu.VMEM((3, HPW, DIM), jnp.float32),
            pltpu.VMEM((2, HPW), jnp.float32),
            pltpu.VMEM((HPW,), jnp.float32),
            pltpu.SemaphoreType.DMA,
            pltpu.SemaphoreType.DMA,
            pltpu.SemaphoreType.DMA,
        ],
        name=f"sc_gather_h{h}",
    )
    def _sc(x_hbm, emb_hbm, lin_hbm, g_hbm, wide_hbm,
            idx_v, rows_v, lvals_v, wsum_v, sem_e, sem_w, sem_l):
        wid = lax.axis_index("s") * 2 + lax.axis_index("c")
        w128 = 16 * h + wid // 2
        off = (wid % 2) * HPW
        lb = (wid // 2) * BPW + off
        pltpu.sync_copy(x_hbm.at[pl.ds(w128 * FIELDS, FIELDS)], idx_v)
        for g in range(HPW // 16):
            wsum_v[pl.ds(g * 16, 16)] = jnp.zeros((16,), jnp.float32)

        def emb_g(f):
            return pltpu.make_async_copy(
                emb_hbm.at[idx_v.at[f, pl.ds(off, HPW)]],
                rows_v.at[f % 3], sem_e)

        def lin_g(f):
            return pltpu.make_async_copy(
                lin_hbm.at[idx_v.at[f, pl.ds(off, HPW)]],
                lvals_v.at[f % 2], sem_l)

        def row_w(f):
            return pltpu.make_async_copy(
                rows_v.at[f % 3],
                g_hbm.at[f // 2, pl.ds(lb, HPW), pl.ds((f % 2) * DIM, DIM)],
                sem_w)

        emb_g(0).start()
        lin_g(0).start()

        def body(f, carry):
            @pl.when(f >= 2)
            def _():
                row_w(f - 2).wait()

            @pl.when(f + 1 < FIELDS)
            def _():
                emb_g(f + 1).start()
                lin_g(f + 1).start()

            emb_g(f).wait()
            row_w(f).start()
            lin_g(f).wait()
            for g in range(HPW // 16):
                sl = pl.ds(g * 16, 16)
                wsum_v[sl] = wsum_v[sl] + lvals_v[f % 2, sl]
            return carry

        lax.fori_loop(0, FIELDS, body, 0)
        row_w(FIELDS - 2).wait()
        row_w(FIELDS - 1).wait()
        pltpu.sync_copy(wsum_v, wide_hbm.at[wid // 2, pl.ds(off, HPW)])

    return _sc


_sc_half = (_make_sc(0), _make_sc(1))


def _mlp_body(g_ref, wv_ref, w1, b1, w2, b2, w3, b3, w4, b4, bias, o_ref):
    h = jnp.dot(g_ref[0].astype(jnp.bfloat16), w1[pl.ds(0, 128), :],
                preferred_element_type=jnp.float32)
    for t in range(1, PAIRS):
        h = h + jnp.dot(g_ref[t].astype(jnp.bfloat16),
                        w1[pl.ds(t * 128, 128), :],
                        preferred_element_type=jnp.float32)
    h = jnp.maximum(h + b1[...], 0.0).astype(jnp.bfloat16)
    h = jnp.dot(h, w2[...], preferred_element_type=jnp.float32)
    h = jnp.maximum(h + b2[...], 0.0).astype(jnp.bfloat16)
    h = jnp.dot(h, w3[...], preferred_element_type=jnp.float32)
    h = jnp.maximum(h + b3[...], 0.0)
    deep = jnp.dot(h, w4[...], preferred_element_type=jnp.float32) + b4[...]
    d2 = jnp.reshape(deep, (8, 128))
    o_ref[...] = jax.nn.sigmoid(d2 + wv_ref[...] + bias[...])


def _mlp(g, wv, W1, b1, W2, b2, W3, b3, W4, b4, bias):
    BLK = 1024
    grid = HALF // BLK
    full2 = lambda shape: pl.BlockSpec(shape, lambda i: (0, 0))
    return pl.pallas_call(
        _mlp_body,
        grid=(grid,),
        in_specs=[
            pl.BlockSpec((PAIRS, BLK, 2 * DIM), lambda i: (0, i, 0)),
            pl.BlockSpec((8, BPW), lambda i: (i, 0)),
            full2(W1.shape), full2(b1.shape),
            full2(W2.shape), full2(b2.shape),
            full2(W3.shape), full2(b3.shape),
            full2(W4.shape), full2(b4.shape),
            full2(bias.shape),
        ],
        out_specs=pl.BlockSpec((8, BPW), lambda i: (i, 0)),
        out_shape=jax.ShapeDtypeStruct((NW // 2, BPW), jnp.float32),
    )(g, wv, W1, b1, W2, b2, W3, b3, W4, b4, bias)


def kernel(x, lin_table, bias, emb_table, W1, b1, W2, b2, W3, b3, W4, b4):
    xt = x.T.reshape(FIELDS, NW, BPW).transpose(1, 0, 2).reshape(
        NW * FIELDS, BPW)
    lin_flat = lin_table.reshape(-1)
    w1b = W1.astype(jnp.bfloat16)
    w2b = W2.astype(jnp.bfloat16)
    w3b = W3.astype(jnp.bfloat16)
    outs = []
    for h in range(2):
        g, wide = _sc_half[h](xt, emb_table, lin_flat)
        outs.append(_mlp(g, wide, w1b, b1.reshape(1, -1),
                         w2b, b2.reshape(1, -1), w3b, b3.reshape(1, -1),
                         W4, b4.reshape(1, -1), bias.reshape(1, 1)))
    return jnp.concatenate(outs, axis=0).reshape(BATCH, 1)


# speedup vs baseline: 1.0543x; 1.0543x over previous
import functools

import jax
import jax.numpy as jnp
from jax import lax
from jax.experimental import pallas as pl
from jax.experimental.pallas import tpu as pltpu
from jax.experimental.pallas import tpu_sc as plsc

BATCH = 4096
FIELDS = 26
PAIRS = FIELDS // 2
DIM = 64
NW = 32
BPW = BATCH // NW

_mesh = plsc.VectorSubcoreMesh(core_axis_name="c", subcore_axis_name="s")

DT_COLS = 2048


def _detile_body(i_ref, o_ref):
    o_ref[:, 0:DIM] = jnp.transpose(i_ref[...])


def _detile(embT):
    grid = -(-100000 // DT_COLS)
    return pl.pallas_call(
        _detile_body,
        grid=(grid,),
        in_specs=[pl.BlockSpec((DIM, DT_COLS), lambda i: (0, i))],
        out_specs=pl.BlockSpec((DT_COLS, 2 * DIM), lambda i: (i, 0)),
        out_shape=jax.ShapeDtypeStruct((100000, 2 * DIM), jnp.float32),
    )(embT)


@functools.partial(
    pl.kernel,
    mesh=_mesh,
    compiler_params=pltpu.CompilerParams(use_tc_tiling_on_sc=False,
                                         needs_layout_passes=False),
    out_type=[
        jax.ShapeDtypeStruct((PAIRS, BATCH, 2 * DIM), jnp.float32),
        jax.ShapeDtypeStruct((NW, BPW), jnp.float32),
    ],
    scratch_types=[
        pltpu.VMEM((FIELDS, BPW), jnp.int32),
        pltpu.VMEM((3, BPW, 2 * DIM), jnp.float32),
        pltpu.VMEM((2, BPW), jnp.float32),
        pltpu.VMEM((BPW,), jnp.float32),
        pltpu.SemaphoreType.DMA,
        pltpu.SemaphoreType.DMA,
        pltpu.SemaphoreType.DMA,
    ],
)
def _sc_gather(x_hbm, emb_hbm, lin_hbm, g_hbm, wide_hbm,
               idx_v, rows_v, lvals_v, wsum_v, sem_e, sem_w, sem_l):
    wid = lax.axis_index("s") * 2 + lax.axis_index("c")
    b0 = wid * BPW
    pltpu.sync_copy(x_hbm.at[pl.ds(wid * FIELDS, FIELDS)], idx_v)
    for g in range(BPW // 16):
        wsum_v[pl.ds(g * 16, 16)] = jnp.zeros((16,), jnp.float32)

    def emb_g(f):
        return pltpu.make_async_copy(
            emb_hbm.at[idx_v.at[f]], rows_v.at[f % 3], sem_e)

    def lin_g(f):
        return pltpu.make_async_copy(
            lin_hbm.at[idx_v.at[f]], lvals_v.at[f % 2], sem_l)

    def row_w(f):
        return pltpu.make_async_copy(
            rows_v.at[f % 3, :, pl.ds(0, DIM)],
            g_hbm.at[f // 2, pl.ds(b0, BPW), pl.ds((f % 2) * DIM, DIM)],
            sem_w)

    emb_g(0).start()
    lin_g(0).start()

    def body(f, carry):
        @pl.when(f >= 2)
        def _():
            row_w(f - 2).wait()

        @pl.when(f + 1 < FIELDS)
        def _():
            emb_g(f + 1).start()
            lin_g(f + 1).start()

        emb_g(f).wait()
        row_w(f).start()
        lin_g(f).wait()
        for g in range(BPW // 16):
            sl = pl.ds(g * 16, 16)
            wsum_v[sl] = wsum_v[sl] + lvals_v[f % 2, sl]
        return carry

    lax.fori_loop(0, FIELDS, body, 0)
    row_w(FIELDS - 2).wait()
    row_w(FIELDS - 1).wait()
    pltpu.sync_copy(wsum_v, wide_hbm.at[wid])


def _mlp_body(g_ref, wv_ref, w1, b1, w2, b2, w3, b3, w4, b4, bias, o_ref):
    h = jnp.dot(g_ref[0].astype(jnp.bfloat16), w1[pl.ds(0, 128), :],
                preferred_element_type=jnp.float32)
    for t in range(1, PAIRS):
        h = h + jnp.dot(g_ref[t].astype(jnp.bfloat16),
                        w1[pl.ds(t * 128, 128), :],
                        preferred_element_type=jnp.float32)
    h = jnp.maximum(h + b1[...], 0.0).astype(jnp.bfloat16)
    h = jnp.dot(h, w2[...], preferred_element_type=jnp.float32)
    h = jnp.maximum(h + b2[...], 0.0).astype(jnp.bfloat16)
    h = jnp.dot(h, w3[...], preferred_element_type=jnp.float32)
    h = jnp.maximum(h + b3[...], 0.0)
    deep = jnp.dot(h, w4[...], preferred_element_type=jnp.float32) + b4[...]
    d2 = jnp.reshape(deep, (8, 128))
    o_ref[...] = jax.nn.sigmoid(d2 + wv_ref[...] + bias[...])


def _mlp(g, wv, W1, b1, W2, b2, W3, b3, W4, b4, bias):
    BLK = 1024
    grid = BATCH // BLK
    full2 = lambda shape: pl.BlockSpec(shape, lambda i: (0, 0))
    return pl.pallas_call(
        _mlp_body,
        grid=(grid,),
        in_specs=[
            pl.BlockSpec((PAIRS, BLK, 2 * DIM), lambda i: (0, i, 0)),
            pl.BlockSpec((8, BPW), lambda i: (i, 0)),
            full2(W1.shape), full2(b1.shape),
            full2(W2.shape), full2(b2.shape),
            full2(W3.shape), full2(b3.shape),
            full2(W4.shape), full2(b4.shape),
            full2(bias.shape),
        ],
        out_specs=pl.BlockSpec((8, BPW), lambda i: (i, 0)),
        out_shape=jax.ShapeDtypeStruct((NW, BPW), jnp.float32),
    )(g, wv, W1, b1, W2, b2, W3, b3, W4, b4, bias)


def kernel(x, lin_table, bias, emb_table, W1, b1, W2, b2, W3, b3, W4, b4):
    xt = x.T.reshape(FIELDS, NW, BPW).transpose(1, 0, 2).reshape(
        NW * FIELDS, BPW)
    lin_flat = lin_table.reshape(-1)
    emb_wide = _detile(emb_table.T)
    g, wide = _sc_gather(xt, emb_wide, lin_flat)
    out2d = _mlp(g, wide,
                 W1.astype(jnp.bfloat16), b1.reshape(1, -1),
                 W2.astype(jnp.bfloat16), b2.reshape(1, -1),
                 W3.astype(jnp.bfloat16), b3.reshape(1, -1),
                 W4, b4.reshape(1, -1),
                 bias.reshape(1, 1))
    return out2d.reshape(BATCH, 1)


# speedup vs baseline: 1.2366x; 1.1729x over previous
import functools

import jax
import jax.numpy as jnp
from jax import lax
from jax.experimental import pallas as pl
from jax.experimental.pallas import tpu as pltpu
from jax.experimental.pallas import tpu_sc as plsc

BATCH = 4096
FIELDS = 26
PAIRS = FIELDS // 2
DIM = 64
NW = 32
BPW = BATCH // NW

_mesh = plsc.VectorSubcoreMesh(core_axis_name="c", subcore_axis_name="s")

DT_COLS = 8192


def _detile_body(i_ref, o_ref):
    o_ref[:, 0:DIM] = jnp.transpose(i_ref[...])


def _detile(embT):
    grid = -(-100000 // DT_COLS)
    return pl.pallas_call(
        _detile_body,
        grid=(grid,),
        in_specs=[pl.BlockSpec((DIM, DT_COLS), lambda i: (0, i))],
        out_specs=pl.BlockSpec((DT_COLS, 2 * DIM), lambda i: (i, 0)),
        out_shape=jax.ShapeDtypeStruct((100000, 2 * DIM), jnp.float32),
    )(embT)


@functools.partial(
    pl.kernel,
    mesh=_mesh,
    compiler_params=pltpu.CompilerParams(use_tc_tiling_on_sc=False,
                                         needs_layout_passes=False),
    out_type=[
        jax.ShapeDtypeStruct((PAIRS, BATCH, 2 * DIM), jnp.float32),
        jax.ShapeDtypeStruct((NW, BPW), jnp.float32),
    ],
    scratch_types=[
        pltpu.VMEM((FIELDS, BPW), jnp.int32),
        pltpu.VMEM((3, BPW, 2 * DIM), jnp.float32),
        pltpu.VMEM((2, BPW), jnp.float32),
        pltpu.VMEM((BPW,), jnp.float32),
        pltpu.SemaphoreType.DMA,
        pltpu.SemaphoreType.DMA,
        pltpu.SemaphoreType.DMA,
    ],
)
def _sc_gather(x_hbm, emb_hbm, lin_hbm, g_hbm, wide_hbm,
               idx_v, rows_v, lvals_v, wsum_v, sem_e, sem_w, sem_l):
    wid = lax.axis_index("s") * 2 + lax.axis_index("c")
    b0 = wid * BPW
    pltpu.sync_copy(x_hbm.at[pl.ds(wid * FIELDS, FIELDS)], idx_v)
    for g in range(BPW // 16):
        wsum_v[pl.ds(g * 16, 16)] = jnp.zeros((16,), jnp.float32)

    def emb_g(f):
        return pltpu.make_async_copy(
            emb_hbm.at[idx_v.at[f]], rows_v.at[f % 3], sem_e)

    def lin_g(f):
        return pltpu.make_async_copy(
            lin_hbm.at[idx_v.at[f]], lvals_v.at[f % 2], sem_l)

    def row_w(f):
        return pltpu.make_async_copy(
            rows_v.at[f % 3, :, pl.ds(0, DIM)],
            g_hbm.at[f // 2, pl.ds(b0, BPW), pl.ds((f % 2) * DIM, DIM)],
            sem_w)

    emb_g(0).start()
    lin_g(0).start()

    def body(f, carry):
        @pl.when(f >= 2)
        def _():
            row_w(f - 2).wait()

        @pl.when(f + 1 < FIELDS)
        def _():
            emb_g(f + 1).start()
            lin_g(f + 1).start()

        emb_g(f).wait()
        row_w(f).start()
        lin_g(f).wait()
        for g in range(BPW // 16):
            sl = pl.ds(g * 16, 16)
            wsum_v[sl] = wsum_v[sl] + lvals_v[f % 2, sl]
        return carry

    lax.fori_loop(0, FIELDS, body, 0)
    row_w(FIELDS - 2).wait()
    row_w(FIELDS - 1).wait()
    pltpu.sync_copy(wsum_v, wide_hbm.at[wid])


def _mlp_body(g_ref, wv_ref, w1, b1, w2, b2, w3, b3, w4, b4, bias, o_ref):
    h = jnp.dot(g_ref[0].astype(jnp.bfloat16), w1[pl.ds(0, 128), :],
                preferred_element_type=jnp.float32)
    for t in range(1, PAIRS):
        h = h + jnp.dot(g_ref[t].astype(jnp.bfloat16),
                        w1[pl.ds(t * 128, 128), :],
                        preferred_element_type=jnp.float32)
    h = jnp.maximum(h + b1[...], 0.0).astype(jnp.bfloat16)
    h = jnp.dot(h, w2[...], preferred_element_type=jnp.float32)
    h = jnp.maximum(h + b2[...], 0.0).astype(jnp.bfloat16)
    h = jnp.dot(h, w3[...], preferred_element_type=jnp.float32)
    h = jnp.maximum(h + b3[...], 0.0)
    deep = jnp.dot(h, w4[...], preferred_element_type=jnp.float32) + b4[...]
    d2 = jnp.reshape(deep, (8, 128))
    o_ref[...] = jax.nn.sigmoid(d2 + wv_ref[...] + bias[...])


def _mlp(g, wv, W1, b1, W2, b2, W3, b3, W4, b4, bias):
    BLK = 1024
    grid = BATCH // BLK
    full2 = lambda shape: pl.BlockSpec(shape, lambda i: (0, 0))
    return pl.pallas_call(
        _mlp_body,
        grid=(grid,),
        in_specs=[
            pl.BlockSpec((PAIRS, BLK, 2 * DIM), lambda i: (0, i, 0)),
            pl.BlockSpec((8, BPW), lambda i: (i, 0)),
            full2(W1.shape), full2(b1.shape),
            full2(W2.shape), full2(b2.shape),
            full2(W3.shape), full2(b3.shape),
            full2(W4.shape), full2(b4.shape),
            full2(bias.shape),
        ],
        out_specs=pl.BlockSpec((8, BPW), lambda i: (i, 0)),
        out_shape=jax.ShapeDtypeStruct((NW, BPW), jnp.float32),
    )(g, wv, W1, b1, W2, b2, W3, b3, W4, b4, bias)


def kernel(x, lin_table, bias, emb_table, W1, b1, W2, b2, W3, b3, W4, b4):
    xt = x.T.reshape(FIELDS, NW, BPW).transpose(1, 0, 2).reshape(
        NW * FIELDS, BPW)
    lin_flat = lin_table.reshape(-1)
    emb_wide = _detile(emb_table.T)
    g, wide = _sc_gather(xt, emb_wide, lin_flat)
    out2d = _mlp(g, wide,
                 W1.astype(jnp.bfloat16), b1.reshape(1, -1),
                 W2.astype(jnp.bfloat16), b2.reshape(1, -1),
                 W3.astype(jnp.bfloat16), b3.reshape(1, -1),
                 W4, b4.reshape(1, -1),
                 bias.reshape(1, 1))
    return out2d.reshape(BATCH, 1)
